# baseline (speedup 1.0000x reference)
import functools
import math

import jax
import jax.numpy as jnp
from jax import lax
from jax.experimental import pallas as pl
from jax.experimental.pallas import tpu as pltpu
from jax.experimental.pallas import tpu_sc as plsc

_VOCAB = 1_000_000
_D = 64
_B = 4096 * 200
_NC, _NS, _L = 2, 16, 16
_NW = _NC * _NS
_B_PER_W = _B // _NW
_CHUNK = 128
_N_CHUNKS = _B_PER_W // _CHUNK
_SCALE = math.sqrt(_D)

_mesh = plsc.VectorSubcoreMesh(core_axis_name="c", subcore_axis_name="s")


@functools.partial(
    pl.kernel,
    out_type=jax.ShapeDtypeStruct((_NW, _N_CHUNKS, _CHUNK, _D), jnp.float32),
    mesh=_mesh,
    scratch_types=[
        pltpu.VMEM((_N_CHUNKS, _CHUNK), jnp.int32),
        pltpu.VMEM((_CHUNK, _D), jnp.float32),
        pltpu.SemaphoreType.DMA,
    ],
    compiler_params=pltpu.CompilerParams(use_tc_tiling_on_sc=False),
)
def _embed_sc(idx_hbm, table_hbm, out_hbm, idx_v, rows_v, gsem):
    wid = lax.axis_index("s") * _NC + lax.axis_index("c")
    pltpu.sync_copy(idx_hbm.at[wid], idx_v)

    def chunk_body(j, _):
        pltpu.async_copy(table_hbm.at[idx_v.at[j]], rows_v, gsem).wait()

        def scale_row(i, _):
            for t in range(_D // _L):
                sl = pl.ds(t * _L, _L)
                rows_v[i, sl] = rows_v[i, sl] * _SCALE
            return 0

        lax.fori_loop(0, _CHUNK, scale_row, 0)

        pltpu.sync_copy(rows_v, out_hbm.at[wid, j])
        return 0

    lax.fori_loop(0, _N_CHUNKS, chunk_body, 0)


def kernel(x, embed):
    idx = x.reshape(_NW, _N_CHUNKS, _CHUNK).astype(jnp.int32)
    out = _embed_sc(idx, embed)
    return out.reshape(x.shape[0], x.shape[1], _D)


# speedup vs baseline: 1.2011x; 1.2011x over previous
import functools
import math

import jax
import jax.numpy as jnp
from jax import lax
from jax.experimental import pallas as pl
from jax.experimental.pallas import tpu as pltpu
from jax.experimental.pallas import tpu_sc as plsc

_D = 64
_B = 4096 * 200
_NC, _NS, _L = 2, 16, 16
_NW = _NC * _NS
_B_PER_W = _B // _NW
_CHUNK = 128
_N_CHUNKS = _B_PER_W // _CHUNK
_NBUF = 4
_N_ROUNDS = _N_CHUNKS // _NBUF
_ROWS_PER_ITER = 8
_SCALE = math.sqrt(_D)

_mesh = plsc.VectorSubcoreMesh(core_axis_name="c", subcore_axis_name="s")


@functools.partial(
    pl.kernel,
    out_type=jax.ShapeDtypeStruct((_NW, _N_CHUNKS, _CHUNK, _D), jnp.float32),
    mesh=_mesh,
    scratch_types=[
        pltpu.VMEM((_N_CHUNKS, _CHUNK), jnp.int32),
        [pltpu.VMEM((_CHUNK, _D), jnp.float32) for _ in range(_NBUF)],
        [pltpu.SemaphoreType.DMA for _ in range(_NBUF)],
        [pltpu.SemaphoreType.DMA for _ in range(_NBUF)],
    ],
    compiler_params=pltpu.CompilerParams(use_tc_tiling_on_sc=False),
)
def _embed_sc(idx_hbm, table_hbm, out_hbm, idx_v, rows, gsems, osems):
    wid = lax.axis_index("s") * _NC + lax.axis_index("c")
    pltpu.sync_copy(idx_hbm.at[wid], idx_v)

    def start_gather(j, b):
        return pltpu.async_copy(table_hbm.at[idx_v.at[j]], rows[b], gsems[b])

    def wait_gather(j, b):
        pltpu.make_async_copy(table_hbm.at[idx_v.at[j]], rows[b], gsems[b]).wait()

    def scale(b):
        def body(i, _):
            r0 = i * _ROWS_PER_ITER
            for k in range(_ROWS_PER_ITER):
                for t in range(_D // _L):
                    sl = pl.ds(t * _L, _L)
                    rows[b][r0 + k, sl] = rows[b][r0 + k, sl] * _SCALE
            return 0

        lax.fori_loop(0, _CHUNK // _ROWS_PER_ITER, body, 0)

    def start_out(j, b):
        return pltpu.async_copy(rows[b], out_hbm.at[wid, j], osems[b])

    def wait_out(j, b):
        pltpu.make_async_copy(rows[b], out_hbm.at[wid, j], osems[b]).wait()

    for b in range(_NBUF):
        start_gather(b, b)

    def round_body(g, _):
        j0 = g * _NBUF
        for b in range(_NBUF):
            wait_gather(j0 + b, b)
            scale(b)
            start_out(j0 + b, b)
        for b in range(_NBUF):
            wait_out(j0 + b, b)
            start_gather(j0 + _NBUF + b, b)
        return 0

    lax.fori_loop(0, _N_ROUNDS - 1, round_body, 0)

    j0 = (_N_ROUNDS - 1) * _NBUF
    for b in range(_NBUF):
        wait_gather(j0 + b, b)
        scale(b)
        start_out(j0 + b, b)
    for b in range(_NBUF):
        wait_out(j0 + b, b)


def kernel(x, embed):
    idx = x.reshape(_NW, _N_CHUNKS, _CHUNK).astype(jnp.int32)
    out = _embed_sc(idx, embed)
    return out.reshape(x.shape[0], x.shape[1], _D)
